# baseline (speedup 1.0000x reference)
import functools

import jax
import jax.numpy as jnp
from jax import lax
from jax.experimental import pallas as pl
from jax.experimental.pallas import tpu as pltpu
from jax.experimental.pallas import tpu_sc as plsc

_NUM_CLASSES = 1000000
_HIDDEN = 64
_BATCH = 16384

_INFO = plsc.get_sparse_core_info()
_NC = _INFO.num_cores
_NS = _INFO.num_subcores
_L = _INFO.num_lanes
_NW = _NC * _NS
_B_PER_W = _BATCH // _NW
_NBLK = (_NUM_CLASSES + 1 + 127) // 128

_mesh = plsc.VectorSubcoreMesh(core_axis_name="c", subcore_axis_name="s")


@functools.partial(
    pl.kernel,
    mesh=_mesh,
    out_type=jax.ShapeDtypeStruct((_BATCH, _HIDDEN), jnp.float32),
    scratch_types=[
        pltpu.SMEM((_B_PER_W,), jnp.int32),
        pltpu.SMEM((_B_PER_W,), jnp.int32),
        pltpu.VMEM_SHARED((_NW, _B_PER_W), jnp.int32),
        pltpu.VMEM_SHARED((_NW, _B_PER_W), jnp.int32),
        pltpu.VMEM((6, _HIDDEN, 128), jnp.float32),
        pltpu.VMEM((_B_PER_W, _HIDDEN), jnp.float32),
        pltpu.SemaphoreType.DMA,
        pltpu.SemaphoreType.DMA,
        pltpu.SemaphoreType.DMA,
        pltpu.SemaphoreType.DMA,
        pltpu.SemaphoreType.DMA,
        pltpu.SemaphoreType.DMA,
    ],
    compiler_params=pltpu.CompilerParams(needs_layout_passes=False),
)
def _embed(labels_hbm, drop_hbm, tab_t_hbm, tail_hbm, out_hbm,
           lbl_s, drop_s, lbl_sp, drop_sp, ring_v, out_v,
           sem0, sem1, sem2, sem3, sem4, sem5):
    wid = lax.axis_index("s") * _NC + lax.axis_index("c")
    base = wid * _B_PER_W
    iota = lax.iota(jnp.int32, _L)
    sems = (sem0, sem1, sem2, sem3, sem4, sem5)

    pltpu.sync_copy(labels_hbm.at[pl.ds(base, _B_PER_W)], lbl_sp.at[wid])
    pltpu.sync_copy(drop_hbm.at[pl.ds(base, _B_PER_W)], drop_sp.at[wid])
    pltpu.sync_copy(lbl_sp.at[wid], lbl_s)
    pltpu.sync_copy(drop_sp.at[wid], drop_s)

    def selected(i):
        return lax.select(drop_s[i] != 0, _NUM_CLASSES, lbl_s[i])

    def issue_fetch(i, par):
        r = selected(i)
        c = r >> 7
        slot = ring_v.at[par]

        @pl.when(c < _NBLK - 1)
        def _main():
            off = pl.multiple_of(c * 128, 128)
            for st in range(_HIDDEN // 8):
                pltpu.async_copy(
                    tab_t_hbm.at[pl.ds(st * 8, 8), pl.ds(off, 128)],
                    slot.at[pl.ds(st * 8, 8)],
                    sems[par])

        @pl.when(c == _NBLK - 1)
        def _tail():
            pltpu.async_copy(tail_hbm, slot, sems[par])

    def wait_fetch(par):
        pltpu.make_async_copy(tail_hbm, ring_v.at[par], sems[par]).wait()

    def extract(i, par):
        r = selected(i)
        lane = jnp.full((_L,), r & 127, jnp.int32)
        slot = ring_v.at[par]
        orow = out_v.at[i]
        for q in range(_HIDDEN // _L):
            hv = iota + q * _L
            x = plsc.load_gather(slot, [hv, lane])
            orow[pl.ds(q * _L, _L)] = x

    for par in range(6):
        issue_fetch(par, par)

    def body(g, _):
        i = g * 6
        for par in range(6):
            wait_fetch(par)
            extract(i + par, par)

            @pl.when(i + par + 6 < _B_PER_W)
            def _next():
                issue_fetch(i + par + 6, par)

        return 0

    lax.fori_loop(0, _B_PER_W // 6, body, 0)
    for k in range(_B_PER_W - 6 * (_B_PER_W // 6)):
        i = 6 * (_B_PER_W // 6) + k
        par = i % 6
        wait_fetch(par)
        extract(i, par)
    pltpu.sync_copy(out_v, out_hbm.at[pl.ds(base, _B_PER_W)])


def kernel(labels, force_drop_ids, embedding_table):
    lbl = labels.astype(jnp.int32)
    drop = force_drop_ids.astype(jnp.int32)
    tab_t = embedding_table.T
    tail_start = (_NBLK - 1) * 128
    tail = jnp.pad(tab_t[:, tail_start:],
                   ((0, 0), (0, _NBLK * 128 - (_NUM_CLASSES + 1))))
    return _embed(lbl, drop, tab_t, tail)
